# baseline (speedup 1.0000x reference)
import functools

import jax
import jax.numpy as jnp
from jax import lax
from jax.experimental import pallas as pl
from jax.experimental.pallas import tpu as pltpu
from jax.experimental.pallas import tpu_sc as plsc

N_ROWS = 320000
D = 128
NUM_SEGMENTS = 1024
CHUNK = 128
N_CHUNKS = N_ROWS // CHUNK
NC = 2
NS = 16
NW = NC * NS
NBUF = 4

_mesh = plsc.VectorSubcoreMesh(core_axis_name="c", subcore_axis_name="s")

_f32 = jnp.float32
_i32 = jnp.int32


@functools.partial(
    pl.kernel,
    out_type=(
        jax.ShapeDtypeStruct((NC * NUM_SEGMENTS, D), _f32),
        jax.ShapeDtypeStruct((NW, NUM_SEGMENTS), _f32),
    ),
    mesh=_mesh,
    compiler_params=pltpu.CompilerParams(needs_layout_passes=False),
    scratch_types=[
        pltpu.VMEM_SHARED((NUM_SEGMENTS, D), _f32),
        *[pltpu.VMEM((CHUNK, D), _f32) for _ in range(NBUF)],
        *[pltpu.VMEM((CHUNK,), _i32) for _ in range(NBUF)],
        pltpu.VMEM((NUM_SEGMENTS,), _f32),
        pltpu.VMEM((16, D), _f32),
        *[pltpu.SemaphoreType.DMA for _ in range(3 * NBUF)],
    ],
)
def _partial_sums(x_hbm, batch_hbm, psum_hbm, pcnt_hbm, acc, *rest):
    xb = rest[0:NBUF]
    ib = rest[NBUF:2 * NBUF]
    cnt = rest[2 * NBUF]
    zrow = rest[2 * NBUF + 1]
    semx = rest[2 * NBUF + 2:3 * NBUF + 2]
    semi = rest[3 * NBUF + 2:4 * NBUF + 2]
    sems = rest[4 * NBUF + 2:5 * NBUF + 2]
    core = lax.axis_index("c")
    sub = lax.axis_index("s")
    wid = core * NS + sub

    z16 = jnp.zeros((16,), _f32)

    def _fill(k, _):
        zrow[k // 8, pl.ds((k % 8) * 16, 16)] = z16
        return 0
    lax.fori_loop(0, CHUNK, _fill, 0)

    def _fillc(k, _):
        cnt[pl.ds(k * 16, 16)] = z16
        return 0
    lax.fori_loop(0, NUM_SEGMENTS // 16, _fillc, 0)

    seg_base = sub * (NUM_SEGMENTS // NS)
    for t in range(NUM_SEGMENTS // NS // 16):
        pltpu.sync_copy(zrow, acc.at[pl.ds(seg_base + t * 16, 16)])
    plsc.subcore_barrier()

    per = N_CHUNKS // NW
    rem = N_CHUNKS - per * NW
    start = wid * per + jnp.minimum(wid, rem)
    n_mine = per + jnp.where(wid < rem, 1, 0)

    def _group(p, _):
        for b in range(NBUF):
            i = NBUF * p + b

            @pl.when(i < n_mine)
            def _gather(i=i, b=b):
                @pl.when(i >= NBUF)
                def _():
                    pltpu.make_async_copy(xb[b], acc.at[ib[b]], sems[b]).wait()
                r0 = (start + i) * CHUNK
                pltpu.async_copy(batch_hbm.at[pl.ds(r0, CHUNK)], ib[b], semi[b])
                pltpu.async_copy(x_hbm.at[pl.ds(r0, CHUNK)], xb[b], semx[b])

            @pl.when(jnp.logical_and(i >= 2, i <= n_mine + 1))
            def _consume(i=i, b=b):
                o = (b - 2) % NBUF
                pltpu.make_async_copy(batch_hbm.at[pl.ds(0, CHUNK)],
                                      ib[o], semi[o]).wait()
                pltpu.make_async_copy(x_hbm.at[pl.ds(0, CHUNK)],
                                      xb[o], semx[o]).wait()
                pltpu.async_copy(xb[o], acc.at[ib[o]], sems[o], add=True)
                for j in range(CHUNK // 16):
                    cur = ib[o][pl.ds(16 * j, 16)]
                    run, last = plsc.scan_count(cur)
                    plsc.addupdate_scatter(cnt, [cur], run.astype(_f32),
                                           mask=last)
        return 0
    n_steps = n_mine + 2
    lax.fori_loop(0, (n_steps + NBUF - 1) // NBUF, _group, 0)

    for k in range(NBUF):
        @pl.when(k < jnp.minimum(n_mine, NBUF))
        def _(k=k):
            pltpu.make_async_copy(xb[k], acc.at[ib[k]], sems[k]).wait()

    plsc.subcore_barrier()

    rows = NUM_SEGMENTS // NS
    out_base = core * NUM_SEGMENTS + sub * rows
    pltpu.sync_copy(acc.at[pl.ds(sub * rows, rows)],
                    psum_hbm.at[pl.ds(out_base, rows)])
    pltpu.sync_copy(cnt, pcnt_hbm.at[wid])


def _combine_body(psum_ref, pcnt_ref, out_ref):
    s = psum_ref[0:NUM_SEGMENTS, :] + psum_ref[NUM_SEGMENTS:2 * NUM_SEGMENTS, :]
    c = jnp.sum(pcnt_ref[...], axis=0)
    out_ref[...] = s / jnp.maximum(c, 1.0)[:, None]


def _combine(psum, pcnt):
    return pl.pallas_call(
        _combine_body,
        out_shape=jax.ShapeDtypeStruct((NUM_SEGMENTS, D), _f32),
    )(psum, pcnt)


def kernel(x, batch):
    batch = batch.astype(jnp.int32)
    psum, pcnt = _partial_sums(x, batch)
    return _combine(psum, pcnt)
